# baseline (speedup 1.0000x reference)
import jax
import jax.numpy as jnp
from jax.experimental import pallas as pl
from jax.experimental.pallas import tpu as pltpu

_N = 8 * 96 * 224 * 224
_LANES = 1024
_ROWS = _N // _LANES
_BM = 256
_GRID = _ROWS // _BM
_SUB = 8
_CHUNKS = _BM // _SUB


def _body(rec_ref, tgt_ref, out_ref, accf_ref, acci_ref):
    step = pl.program_id(0)

    @pl.when(step == 0)
    def _init():
        accf_ref[...] = jnp.zeros_like(accf_ref)
        acci_ref[...] = jnp.zeros_like(acci_ref)

    for c in range(_CHUNKS):
        r = rec_ref[c * _SUB:(c + 1) * _SUB, :]
        t = tgt_ref[c * _SUB:(c + 1) * _SUB, :]
        d = r - t
        sq = d * d
        zm = t == 0.0
        rz = r == 0.0
        tn = zm & rz
        fn = tn != rz
        tm = tn != (r == t)
        zero_f = jnp.zeros_like(sq)
        one_i = jnp.ones(sq.shape, jnp.int32)
        zero_i = jnp.zeros(sq.shape, jnp.int32)
        accf_ref[0] += sq
        accf_ref[1] += jnp.where(zm, sq, zero_f)
        accf_ref[2] += jnp.where(fn, sq, zero_f)
        acci_ref[0] += jnp.where(zm, one_i, zero_i)
        acci_ref[1] += jnp.where(tn, one_i, zero_i)
        acci_ref[2] += jnp.where(fn, one_i, zero_i)
        acci_ref[3] += jnp.where(tm, one_i, zero_i)

    @pl.when(step == _GRID - 1)
    def _final():
        s_all = jnp.sum(accf_ref[0])
        s_z = jnp.sum(accf_ref[1])
        s_fn = jnp.sum(accf_ref[2])
        c_z = jnp.sum(acci_ref[0])
        c_tn = jnp.sum(acci_ref[1])
        c_fn = jnp.sum(acci_ref[2])
        c_tm = jnp.sum(acci_ref[3])

        n_f = jnp.float32(_N)
        c_nz = _N - c_z
        s_nz = s_all - s_z
        c_tp = c_nz - c_fn
        s_tp = s_nz - s_fn
        c_fp = c_z - c_tn

        def mse(s, c, repl):
            m = s / jnp.maximum(c, 1).astype(jnp.float32)
            return jnp.where(c == 0, jnp.float32(repl), m)

        ff_loss = s_all / n_f
        zero_loss = mse(s_z, c_z, 0.0)
        nonzero_loss = mse(s_nz, c_nz, 0.0)
        time_match = jnp.where(c_tm == 0, jnp.float32(10.0), jnp.float32(0.0))
        fnl = mse(s_fn, c_fn, 0.0)
        fpl = mse(s_tp, c_tp, 0.0)
        tnl = jnp.where(c_tn == 0, jnp.float32(10.0), jnp.float32(0.0))
        tpl = mse(s_z, c_fp, 10.0)

        out_ref[0, 0] = (tpl + fnl + fpl + tnl + time_match
                         + ff_loss + zero_loss + nonzero_loss)


def _run(rec2d, tgt2d, interpret=False):
    return pl.pallas_call(
        _body,
        grid=(_GRID,),
        in_specs=[
            pl.BlockSpec((_BM, _LANES), lambda i: (i, 0)),
            pl.BlockSpec((_BM, _LANES), lambda i: (i, 0)),
        ],
        out_specs=pl.BlockSpec(memory_space=pltpu.SMEM),
        out_shape=jax.ShapeDtypeStruct((1, 1), jnp.float32),
        scratch_shapes=[
            pltpu.VMEM((3, _SUB, _LANES), jnp.float32),
            pltpu.VMEM((4, _SUB, _LANES), jnp.int32),
        ],
        compiler_params=pltpu.CompilerParams(
            dimension_semantics=("arbitrary",),
        ),
        interpret=interpret,
    )(rec2d, tgt2d)


def kernel(reconstructed_image, target_image):
    rec2d = reconstructed_image.reshape(_ROWS, _LANES)
    tgt2d = target_image.reshape(_ROWS, _LANES)
    return _run(rec2d, tgt2d)[0, 0]


# speedup vs baseline: 1.0938x; 1.0938x over previous
import jax
import jax.numpy as jnp
from jax.experimental import pallas as pl
from jax.experimental.pallas import tpu as pltpu

_N = 8 * 96 * 224 * 224
_LANES = 1024
_ROWS = _N // _LANES
_BM = 768
_GRID = _ROWS // _BM
_SUB = 8
_CHUNKS = _BM // _SUB


def _body(rec_ref, tgt_ref, out_ref, accf_ref, acci_ref):
    step = pl.program_id(0)

    @pl.when(step == 0)
    def _init():
        accf_ref[...] = jnp.zeros_like(accf_ref)
        acci_ref[...] = jnp.zeros_like(acci_ref)

    for c in range(_CHUNKS):
        r = rec_ref[c * _SUB:(c + 1) * _SUB, :]
        t = tgt_ref[c * _SUB:(c + 1) * _SUB, :]
        d = r - t
        sq = d * d
        zm = t == 0.0
        rz = r == 0.0
        tn = zm & rz
        fn = tn != rz
        tm = tn != (r == t)
        zero_f = jnp.zeros_like(sq)
        one_i = jnp.ones(sq.shape, jnp.int32)
        zero_i = jnp.zeros(sq.shape, jnp.int32)
        accf_ref[0] += sq
        accf_ref[1] += jnp.where(zm, sq, zero_f)
        accf_ref[2] += jnp.where(fn, sq, zero_f)
        acci_ref[0] += jnp.where(zm, one_i, zero_i)
        acci_ref[1] += jnp.where(tn, one_i, zero_i)
        acci_ref[2] += jnp.where(fn, one_i, zero_i)
        acci_ref[3] += jnp.where(tm, one_i, zero_i)

    @pl.when(step == _GRID - 1)
    def _final():
        s_all = jnp.sum(accf_ref[0])
        s_z = jnp.sum(accf_ref[1])
        s_fn = jnp.sum(accf_ref[2])
        c_z = jnp.sum(acci_ref[0])
        c_tn = jnp.sum(acci_ref[1])
        c_fn = jnp.sum(acci_ref[2])
        c_tm = jnp.sum(acci_ref[3])

        n_f = jnp.float32(_N)
        c_nz = _N - c_z
        s_nz = s_all - s_z
        c_tp = c_nz - c_fn
        s_tp = s_nz - s_fn
        c_fp = c_z - c_tn

        def mse(s, c, repl):
            m = s / jnp.maximum(c, 1).astype(jnp.float32)
            return jnp.where(c == 0, jnp.float32(repl), m)

        ff_loss = s_all / n_f
        zero_loss = mse(s_z, c_z, 0.0)
        nonzero_loss = mse(s_nz, c_nz, 0.0)
        time_match = jnp.where(c_tm == 0, jnp.float32(10.0), jnp.float32(0.0))
        fnl = mse(s_fn, c_fn, 0.0)
        fpl = mse(s_tp, c_tp, 0.0)
        tnl = jnp.where(c_tn == 0, jnp.float32(10.0), jnp.float32(0.0))
        tpl = mse(s_z, c_fp, 10.0)

        out_ref[0, 0] = (tpl + fnl + fpl + tnl + time_match
                         + ff_loss + zero_loss + nonzero_loss)


def _run(rec2d, tgt2d, interpret=False):
    return pl.pallas_call(
        _body,
        grid=(_GRID,),
        in_specs=[
            pl.BlockSpec((_BM, _LANES), lambda i: (i, 0)),
            pl.BlockSpec((_BM, _LANES), lambda i: (i, 0)),
        ],
        out_specs=pl.BlockSpec(memory_space=pltpu.SMEM),
        out_shape=jax.ShapeDtypeStruct((1, 1), jnp.float32),
        scratch_shapes=[
            pltpu.VMEM((3, _SUB, _LANES), jnp.float32),
            pltpu.VMEM((4, _SUB, _LANES), jnp.int32),
        ],
        compiler_params=pltpu.CompilerParams(
            dimension_semantics=("arbitrary",),
        ),
        interpret=interpret,
    )(rec2d, tgt2d)


def kernel(reconstructed_image, target_image):
    rec2d = reconstructed_image.reshape(_ROWS, _LANES)
    tgt2d = target_image.reshape(_ROWS, _LANES)
    return _run(rec2d, tgt2d)[0, 0]


# speedup vs baseline: 1.1134x; 1.0178x over previous
import jax
import jax.numpy as jnp
from jax.experimental import pallas as pl
from jax.experimental.pallas import tpu as pltpu

_N = 8 * 96 * 224 * 224
_LANES = 1024
_ROWS = _N // _LANES
_BM = 1344
_GRID = _ROWS // _BM
_SUB = 8
_CHUNKS = _BM // _SUB


def _body(rec_ref, tgt_ref, out_ref, accf_ref, acci_ref):
    step = pl.program_id(0)

    @pl.when(step == 0)
    def _init():
        accf_ref[...] = jnp.zeros_like(accf_ref)
        acci_ref[...] = jnp.zeros_like(acci_ref)

    for c in range(_CHUNKS):
        r = rec_ref[c * _SUB:(c + 1) * _SUB, :]
        t = tgt_ref[c * _SUB:(c + 1) * _SUB, :]
        d = r - t
        sq = d * d
        zm = t == 0.0
        rz = r == 0.0
        tn = zm & rz
        fn = tn != rz
        tm = tn != (r == t)
        zero_f = jnp.zeros_like(sq)
        one_i = jnp.ones(sq.shape, jnp.int32)
        zero_i = jnp.zeros(sq.shape, jnp.int32)
        accf_ref[0] += sq
        accf_ref[1] += jnp.where(zm, sq, zero_f)
        accf_ref[2] += jnp.where(fn, sq, zero_f)
        acci_ref[0] += jnp.where(zm, one_i, zero_i)
        acci_ref[1] += jnp.where(tn, one_i, zero_i)
        acci_ref[2] += jnp.where(fn, one_i, zero_i)
        acci_ref[3] += jnp.where(tm, one_i, zero_i)

    @pl.when(step == _GRID - 1)
    def _final():
        s_all = jnp.sum(accf_ref[0])
        s_z = jnp.sum(accf_ref[1])
        s_fn = jnp.sum(accf_ref[2])
        c_z = jnp.sum(acci_ref[0])
        c_tn = jnp.sum(acci_ref[1])
        c_fn = jnp.sum(acci_ref[2])
        c_tm = jnp.sum(acci_ref[3])

        n_f = jnp.float32(_N)
        c_nz = _N - c_z
        s_nz = s_all - s_z
        c_tp = c_nz - c_fn
        s_tp = s_nz - s_fn
        c_fp = c_z - c_tn

        def mse(s, c, repl):
            m = s / jnp.maximum(c, 1).astype(jnp.float32)
            return jnp.where(c == 0, jnp.float32(repl), m)

        ff_loss = s_all / n_f
        zero_loss = mse(s_z, c_z, 0.0)
        nonzero_loss = mse(s_nz, c_nz, 0.0)
        time_match = jnp.where(c_tm == 0, jnp.float32(10.0), jnp.float32(0.0))
        fnl = mse(s_fn, c_fn, 0.0)
        fpl = mse(s_tp, c_tp, 0.0)
        tnl = jnp.where(c_tn == 0, jnp.float32(10.0), jnp.float32(0.0))
        tpl = mse(s_z, c_fp, 10.0)

        out_ref[0, 0] = (tpl + fnl + fpl + tnl + time_match
                         + ff_loss + zero_loss + nonzero_loss)


def _run(rec2d, tgt2d, interpret=False):
    return pl.pallas_call(
        _body,
        grid=(_GRID,),
        in_specs=[
            pl.BlockSpec((_BM, _LANES), lambda i: (i, 0)),
            pl.BlockSpec((_BM, _LANES), lambda i: (i, 0)),
        ],
        out_specs=pl.BlockSpec(memory_space=pltpu.SMEM),
        out_shape=jax.ShapeDtypeStruct((1, 1), jnp.float32),
        scratch_shapes=[
            pltpu.VMEM((3, _SUB, _LANES), jnp.float32),
            pltpu.VMEM((4, _SUB, _LANES), jnp.int32),
        ],
        compiler_params=pltpu.CompilerParams(
            dimension_semantics=("arbitrary",),
        ),
        interpret=interpret,
    )(rec2d, tgt2d)


def kernel(reconstructed_image, target_image):
    rec2d = reconstructed_image.reshape(_ROWS, _LANES)
    tgt2d = target_image.reshape(_ROWS, _LANES)
    return _run(rec2d, tgt2d)[0, 0]
